# baseline (device time: 49993 ns/iter reference)
import jax
import jax.numpy as jnp
from jax import lax
from jax.experimental import pallas as pl
from jax.experimental.pallas import tpu as pltpu

N_DEV = 16
SQ = 512
D_MODEL = 1024
N_HEADS = 8
DH = 128
SCALE = 0.08838834764831843
CHUNK = SQ // N_DEV
N_BLK = 4
BLK = SQ // N_BLK
CPB = N_DEV // N_BLK


def kernel(x, Wq, Wo, Wk, Wv):
    def body(x_ref, wq_ref, wk_ref, wv_ref, wo_ref, out_ref,
             send_buf, rs_buf, bc_src, bc_buf,
             rs_send, rs_recv, bc_send, bc_recv, dummy_sem):
        my_d = lax.axis_index("i")
        my_blk = my_d // CPB

        xb = x_ref[...].astype(jnp.bfloat16)
        wq_b = wq_ref[...].astype(jnp.bfloat16)
        wo_b = wo_ref[...].astype(jnp.bfloat16)
        k = jnp.dot(xb, wk_ref[...].astype(jnp.bfloat16),
                    preferred_element_type=jnp.float32)
        v = jnp.dot(xb, wv_ref[...].astype(jnp.bfloat16),
                    preferred_element_type=jnp.float32)

        barrier = pltpu.get_barrier_semaphore()
        for o in range(1, N_DEV):
            peer = jnp.mod(my_d + o, N_DEV)
            pl.semaphore_signal(barrier, inc=1, device_id=(peer,),
                                device_id_type=pl.DeviceIdType.MESH)
        pl.semaphore_wait(barrier, N_DEV - 1)

        kh_b = [k[:, h * DH:(h + 1) * DH].astype(jnp.bfloat16)
                for h in range(N_HEADS)]
        vh_b = [v[:, h * DH:(h + 1) * DH].astype(jnp.bfloat16)
                for h in range(N_HEADS)]

        started = []
        for t in range(N_BLK):
            rbt = jnp.mod(my_blk + t, N_BLK)
            row0 = rbt * BLK
            xq = x_ref[pl.ds(row0, BLK), :].astype(jnp.bfloat16)
            qb = jnp.dot(xq, wq_b, preferred_element_type=jnp.float32)
            attn_cols = []
            for h in range(N_HEADS):
                qh = qb[:, h * DH:(h + 1) * DH].astype(jnp.bfloat16)
                s = lax.dot_general(qh, kh_b[h], (((1,), (1,)), ((), ())),
                                    preferred_element_type=jnp.float32) * SCALE
                m = jnp.max(s, axis=1, keepdims=True)
                p = jnp.exp(s - m)
                l = jnp.sum(p, axis=1, keepdims=True)
                o = jnp.dot(p.astype(jnp.bfloat16), vh_b[h],
                            preferred_element_type=jnp.float32) / l
                attn_cols.append(o.astype(jnp.bfloat16))
            attn_b = jnp.concatenate(attn_cols, axis=1)
            pb = jnp.dot(attn_b, wo_b, preferred_element_type=jnp.float32)
            out_ref[pl.ds(row0, BLK), :] = pb
            send_buf[pl.ds(row0, BLK), :] = pb.astype(jnp.bfloat16)

            for u in range(CPB):
                c = rbt * CPB + u
                slot = jnp.minimum(jnp.mod(my_d - c - 1, N_DEV), N_DEV - 2)
                rdma = pltpu.make_async_remote_copy(
                    src_ref=send_buf.at[pl.ds(c * CHUNK, CHUNK), :],
                    dst_ref=rs_buf.at[pl.ds(slot * CHUNK, CHUNK), :],
                    send_sem=rs_send.at[t * CPB + u],
                    recv_sem=rs_recv.at[slot],
                    device_id=(c,),
                    device_id_type=pl.DeviceIdType.MESH,
                )
                not_self = c != my_d

                @pl.when(not_self)
                def _():
                    rdma.start()

                started.append((rdma, not_self))

        for j in range(N_DEV - 1):
            recv = pltpu.make_async_remote_copy(
                src_ref=rs_buf.at[pl.ds(j * CHUNK, CHUNK), :],
                dst_ref=rs_buf.at[pl.ds(j * CHUNK, CHUNK), :],
                send_sem=dummy_sem.at[0],
                recv_sem=rs_recv.at[j],
                device_id=(my_d,),
                device_id_type=pl.DeviceIdType.MESH,
            )
            recv.wait_recv()

        acc = out_ref[pl.ds(my_d * CHUNK, CHUNK), :]
        acc = acc + jnp.sum(
            rs_buf[...].reshape(N_DEV - 1, CHUNK, D_MODEL).astype(jnp.float32),
            axis=0)
        out_ref[pl.ds(my_d * CHUNK, CHUNK), :] = acc
        bc_src[...] = acc.astype(jnp.bfloat16)

        bc = []
        for o in range(1, N_DEV):
            peer = jnp.mod(my_d + o, N_DEV)
            rdma = pltpu.make_async_remote_copy(
                src_ref=bc_src,
                dst_ref=bc_buf.at[N_DEV - 1 - o],
                send_sem=bc_send.at[o - 1],
                recv_sem=bc_recv.at[N_DEV - 1 - o],
                device_id=(peer,),
                device_id_type=pl.DeviceIdType.MESH,
            )
            rdma.start()
            bc.append(rdma)

        for j in range(N_DEV - 1):
            bc[N_DEV - 2 - j].wait_recv()
            c = jnp.mod(my_d + j + 1, N_DEV)
            out_ref[pl.ds(c * CHUNK, CHUNK), :] = bc_buf[j].astype(jnp.float32)

        for rdma, not_self in started:
            @pl.when(not_self)
            def _():
                rdma.wait_send()
        for rdma in bc:
            rdma.wait_send()

    out = pl.pallas_call(
        body,
        out_shape=jax.ShapeDtypeStruct((SQ, D_MODEL), jnp.float32),
        in_specs=[pl.BlockSpec(memory_space=pltpu.VMEM)] * 5,
        out_specs=pl.BlockSpec(memory_space=pltpu.VMEM),
        scratch_shapes=[
            pltpu.VMEM((SQ, D_MODEL), jnp.bfloat16),
            pltpu.VMEM(((N_DEV - 1) * CHUNK, D_MODEL), jnp.bfloat16),
            pltpu.VMEM((CHUNK, D_MODEL), jnp.bfloat16),
            pltpu.VMEM((N_DEV - 1, CHUNK, D_MODEL), jnp.bfloat16),
            pltpu.SemaphoreType.DMA((N_DEV,)),
            pltpu.SemaphoreType.DMA((N_DEV - 1,)),
            pltpu.SemaphoreType.DMA((N_DEV - 1,)),
            pltpu.SemaphoreType.DMA((N_DEV - 1,)),
            pltpu.SemaphoreType.DMA((1,)),
        ],
        compiler_params=pltpu.CompilerParams(collective_id=0),
    )(x.reshape(SQ, D_MODEL), Wq, Wk, Wv, Wo)
    return out.reshape(1, SQ, D_MODEL)
